# baseline (speedup 1.0000x reference)
import functools

import jax
import jax.numpy as jnp
from jax import lax
from jax.experimental import pallas as pl
from jax.experimental.pallas import tpu as pltpu
from jax.experimental.pallas import tpu_sc as plsc

B = 16384
N_COV = 128
N_PER = 1024
NC = 2
NS = 16
NW = NC * NS
BPW = B // NW
LANES = 16


def _sc_fe_body(eid_hbm, tid_hbm, efe_hbm, tfe_hbm, out_hbm,
                eidx_v, tidx_v, e_v, t_v,
                sem_ei, sem_ti, sem_t, sem_e):
    wid = lax.axis_index("s") * NC + lax.axis_index("c")
    base = wid * BPW
    cp_ei = pltpu.async_copy(eid_hbm.at[pl.ds(base, BPW)], eidx_v, sem_ei)
    cp_ti = pltpu.async_copy(tid_hbm.at[pl.ds(base, BPW)], tidx_v, sem_ti)
    cp_ei.wait()
    cp_e = pltpu.async_copy(efe_hbm.at[eidx_v], e_v, sem_e)
    cp_ti.wait()
    cp_t = pltpu.async_copy(tfe_hbm.at[tidx_v], t_v, sem_t)
    cp_t.wait()
    cp_e.wait()
    for i in range(BPW // LANES):
        sl = pl.ds(i * LANES, LANES)
        e_v[sl] = e_v[sl] + t_v[sl]
    pltpu.sync_copy(e_v, out_hbm.at[pl.ds(base, BPW)])


_sc_fe = functools.partial(
    pl.kernel,
    mesh=plsc.VectorSubcoreMesh(core_axis_name="c", subcore_axis_name="s"),
    out_type=jax.ShapeDtypeStruct((B,), jnp.float32),
    scratch_types=[
        pltpu.VMEM((BPW,), jnp.int32),
        pltpu.VMEM((BPW,), jnp.int32),
        pltpu.VMEM((BPW,), jnp.float32),
        pltpu.VMEM((BPW,), jnp.float32),
        pltpu.SemaphoreType.DMA,
        pltpu.SemaphoreType.DMA,
        pltpu.SemaphoreType.DMA,
        pltpu.SemaphoreType.DMA,
    ],
)(_sc_fe_body)


def _tc_mv_body(x_ref, w_ref, o_ref):
    o_ref[...] = jax.lax.dot_general(
        x_ref[...], w_ref[...], (((1,), (1,)), ((), ())),
        preferred_element_type=jnp.float32)


def _tc_matvec(X, beta_w):
    blk = 16384
    return pl.pallas_call(
        _tc_mv_body,
        grid=(B // blk,),
        in_specs=[
            pl.BlockSpec((blk, N_COV), lambda i: (i, 0)),
            pl.BlockSpec((1, N_COV), lambda i: (0, 0)),
        ],
        out_specs=pl.BlockSpec((blk, 1), lambda i: (i, 0)),
        out_shape=jax.ShapeDtypeStruct((B, 1), jnp.float32),
    )(X, beta_w)


def _tc_add_body(a_ref, b_ref, o_ref):
    o_ref[...] = a_ref[...] + b_ref[...]


def _tc_add(a, b):
    return pl.pallas_call(
        _tc_add_body,
        out_shape=jax.ShapeDtypeStruct((B,), jnp.float32),
    )(a, b)


@jax.jit
def kernel(entity_ids, time_ids, X, entity_fe, time_fe, beta_w):
    fe_sum = _sc_fe(entity_ids, time_ids,
                    entity_fe.reshape(-1), time_fe.reshape(-1))
    xb = _tc_matvec(X, beta_w)
    return _tc_add(xb.reshape(B), fe_sum)


# speedup vs baseline: 1.0072x; 1.0072x over previous
import functools

import jax
import jax.numpy as jnp
from jax import lax
from jax.experimental import pallas as pl
from jax.experimental.pallas import tpu as pltpu
from jax.experimental.pallas import tpu_sc as plsc

B = 16384
N_COV = 128
N_PER = 1024
NC = 2
NS = 16
NW = NC * NS
BPW = B // NW
LANES = 16


def _sc_fe_body(eid_hbm, tid_hbm, efe_hbm, tfe_hbm, out_hbm,
                eidx_v, tidx_v, e_v, t_v,
                sem_ei, sem_ti, sem_t, sem_e):
    wid = lax.axis_index("s") * NC + lax.axis_index("c")
    base = wid * BPW
    cp_ei = pltpu.async_copy(eid_hbm.at[pl.ds(base, BPW)], eidx_v, sem_ei)
    cp_ti = pltpu.async_copy(tid_hbm.at[pl.ds(base, BPW)], tidx_v, sem_ti)
    cp_ei.wait()
    cp_e = pltpu.async_copy(efe_hbm.at[eidx_v], e_v, sem_e)
    cp_ti.wait()
    cp_t = pltpu.async_copy(tfe_hbm.at[tidx_v], t_v, sem_t)
    cp_t.wait()
    cp_e.wait()
    for i in range(BPW // LANES):
        sl = pl.ds(i * LANES, LANES)
        e_v[sl] = e_v[sl] + t_v[sl]
    pltpu.sync_copy(e_v, out_hbm.at[pl.ds(base, BPW)])


_sc_fe = functools.partial(
    pl.kernel,
    mesh=plsc.VectorSubcoreMesh(core_axis_name="c", subcore_axis_name="s"),
    out_type=jax.ShapeDtypeStruct((B,), jnp.float32),
    scratch_types=[
        pltpu.VMEM((BPW,), jnp.int32),
        pltpu.VMEM((BPW,), jnp.int32),
        pltpu.VMEM((BPW,), jnp.float32),
        pltpu.VMEM((BPW,), jnp.float32),
        pltpu.SemaphoreType.DMA,
        pltpu.SemaphoreType.DMA,
        pltpu.SemaphoreType.DMA,
        pltpu.SemaphoreType.DMA,
    ],
)(_sc_fe_body)


def _tc_mv_body(x_ref, w_ref, o_ref):
    o_ref[...] = jax.lax.dot_general(
        x_ref[...], w_ref[...], (((1,), (1,)), ((), ())),
        preferred_element_type=jnp.float32)


def _tc_matvec(X, beta_w):
    blk = 8192
    return pl.pallas_call(
        _tc_mv_body,
        grid=(B // blk,),
        in_specs=[
            pl.BlockSpec((blk, N_COV), lambda i: (i, 0)),
            pl.BlockSpec((1, N_COV), lambda i: (0, 0)),
        ],
        out_specs=pl.BlockSpec((blk, 1), lambda i: (i, 0)),
        out_shape=jax.ShapeDtypeStruct((B, 1), jnp.float32),
    )(X, beta_w)


def _tc_add_body(a_ref, b_ref, o_ref):
    o_ref[...] = a_ref[...] + b_ref[...]


def _tc_add(a, b):
    return pl.pallas_call(
        _tc_add_body,
        out_shape=jax.ShapeDtypeStruct((B,), jnp.float32),
    )(a, b)


@jax.jit
def kernel(entity_ids, time_ids, X, entity_fe, time_fe, beta_w):
    fe_sum = _sc_fe(entity_ids, time_ids,
                    entity_fe[:, 0], time_fe.reshape(-1))
    xb = _tc_matvec(X, beta_w)
    return _tc_add(xb.reshape(B), fe_sum)


# speedup vs baseline: 1.0107x; 1.0034x over previous
import functools

import jax
import jax.numpy as jnp
from jax import lax
from jax.experimental import pallas as pl
from jax.experimental.pallas import tpu as pltpu
from jax.experimental.pallas import tpu_sc as plsc

B = 16384
N_COV = 128
NC = 2
NS = 16
NW = NC * NS
BPW = B // NW
LANES = 16


def _sc_fe_body(eid_hbm, tid_hbm, efe_hbm, tfe_hbm, out_hbm,
                eidx_v, tidx_v, e_v, t_v,
                sem_ei, sem_ti, sem_t, sem_e):
    wid = lax.axis_index("s") * NC + lax.axis_index("c")
    base = wid * BPW
    cp_ei = pltpu.async_copy(eid_hbm.at[pl.ds(base, BPW)], eidx_v, sem_ei)
    cp_ti = pltpu.async_copy(tid_hbm.at[pl.ds(base, BPW)], tidx_v, sem_ti)
    cp_ei.wait()
    cp_e = pltpu.async_copy(efe_hbm.at[eidx_v], e_v, sem_e)
    cp_ti.wait()
    cp_t = pltpu.async_copy(tfe_hbm.at[tidx_v], t_v, sem_t)
    cp_t.wait()
    cp_e.wait()
    for i in range(BPW // LANES):
        sl = pl.ds(i * LANES, LANES)
        e_v[sl] = e_v[sl] + t_v[sl]
    pltpu.sync_copy(e_v, out_hbm.at[pl.ds(base, BPW)])


_sc_fe = functools.partial(
    pl.kernel,
    mesh=plsc.VectorSubcoreMesh(core_axis_name="c", subcore_axis_name="s"),
    out_type=jax.ShapeDtypeStruct((B,), jnp.float32),
    scratch_types=[
        pltpu.VMEM((BPW,), jnp.int32),
        pltpu.VMEM((BPW,), jnp.int32),
        pltpu.VMEM((BPW,), jnp.float32),
        pltpu.VMEM((BPW,), jnp.float32),
        pltpu.SemaphoreType.DMA,
        pltpu.SemaphoreType.DMA,
        pltpu.SemaphoreType.DMA,
        pltpu.SemaphoreType.DMA,
    ],
)(_sc_fe_body)


def _tc_mv_body(x_ref, w_ref, o_ref):
    o_ref[...] = jax.lax.dot_general(
        x_ref[...], w_ref[...], (((1,), (1,)), ((), ())),
        preferred_element_type=jnp.float32)


def _tc_matvec(X, beta_w):
    blk = 8192
    return pl.pallas_call(
        _tc_mv_body,
        grid=(B // blk,),
        in_specs=[
            pl.BlockSpec((blk, N_COV), lambda i: (i, 0)),
            pl.BlockSpec((1, N_COV), lambda i: (0, 0)),
        ],
        out_specs=pl.BlockSpec((blk, 1), lambda i: (i, 0)),
        out_shape=jax.ShapeDtypeStruct((B, 1), jnp.float32),
    )(X, beta_w)


def _tc_add_body(a_ref, b_ref, o_ref):
    o_ref[...] = a_ref[...] + b_ref[...]


def _tc_add(a, b):
    return pl.pallas_call(
        _tc_add_body,
        out_shape=jax.ShapeDtypeStruct((B,), jnp.float32),
    )(a, b)


@jax.jit
def kernel(entity_ids, time_ids, X, entity_fe, time_fe, beta_w):
    fe_sum = _sc_fe(entity_ids, time_ids,
                    entity_fe.reshape(-1), time_fe.reshape(-1))
    xb = _tc_matvec(X, beta_w)
    return _tc_add(xb.reshape(B), fe_sum)
